# baseline (speedup 1.0000x reference)
import jax
import jax.numpy as jnp
from jax.experimental import pallas as pl

N = 8192
D = 256
RB = 256
K = 1024


def _gram_rowmax_body(xa_ref, xb_ref, wcol_ref, wrow_ref, m_ref):
    i = pl.program_id(0)
    P = jax.lax.dot_general(
        xa_ref[...], xb_ref[...], (((1,), (1,)), ((), ())),
        preferred_element_type=jnp.float32)
    G = P / (wcol_ref[...] * wrow_ref[...])
    r = jax.lax.broadcasted_iota(jnp.int32, (RB, N), 0) + i * RB
    c = jax.lax.broadcasted_iota(jnp.int32, (RB, N), 1)
    G = jnp.where(r == c, -jnp.inf, G)
    m_ref[...] = jnp.max(G, axis=1, keepdims=True)


def _rowmax(x, w_col, w_row):
    return pl.pallas_call(
        _gram_rowmax_body,
        grid=(N // RB,),
        in_specs=[
            pl.BlockSpec((RB, D), lambda i: (i, 0)),
            pl.BlockSpec((N, D), lambda i: (0, 0)),
            pl.BlockSpec((RB, 1), lambda i: (i, 0)),
            pl.BlockSpec((1, N), lambda i: (0, 0)),
        ],
        out_specs=pl.BlockSpec((RB, 1), lambda i: (i, 0)),
        out_shape=jax.ShapeDtypeStruct((N, 1), jnp.float32),
    )(x, x, w_col, w_row)


def kernel(x, nb_selected):
    w = jnp.sqrt(jnp.sum(x * x, axis=1, keepdims=True))
    m = _rowmax(x, w, w.reshape(1, N))[:, 0]
    values, inds = jax.lax.top_k(m, K)
    return values, inds
